# baseline (speedup 1.0000x reference)
import functools

import jax
import jax.numpy as jnp
from jax import lax
from jax.experimental import pallas as pl
from jax.experimental.pallas import tpu as pltpu
from jax.experimental.pallas import tpu_sc as plsc

_NC = 2
_NS = 16
_NW = _NC * _NS
_L = 16


def _build_mask_sc(num_rows: int, num_idx: int):
    slab = ((num_rows + _NW - 1) // _NW + _L - 1) // _L * _L
    m_pad = slab * _NW
    mesh = plsc.VectorSubcoreMesh(core_axis_name="c", subcore_axis_name="s")

    @functools.partial(
        pl.kernel,
        out_type=jax.ShapeDtypeStruct((m_pad,), jnp.float32),
        mesh=mesh,
        scratch_types=[
            pltpu.VMEM((num_idx,), jnp.int32),
            pltpu.VMEM((slab,), jnp.float32),
        ],
        compiler_params=pltpu.CompilerParams(needs_layout_passes=False),
    )
    def mask_kernel(idx_hbm, mask_hbm, idx_v, slab_v):
        wid = lax.axis_index("s") * _NC + lax.axis_index("c")
        lo = wid * slab
        pltpu.sync_copy(idx_hbm, idx_v)

        zeros = jnp.zeros((_L,), jnp.float32)

        def zero_body(i, carry):
            slab_v[pl.ds(i * _L, _L)] = zeros
            return carry

        lax.fori_loop(0, slab // _L, zero_body, 0)

        ones = jnp.ones((_L,), jnp.float32)

        def scan_body(i, carry):
            v = idx_v[pl.ds(i * _L, _L)]
            m = (v >= lo) & (v < lo + slab)
            rel = jnp.where(m, v - lo, 0)
            plsc.store_scatter(slab_v, (rel,), ones, mask=m)
            return carry

        lax.fori_loop(0, num_idx // _L, scan_body, 0)

        pltpu.sync_copy(slab_v, mask_hbm.at[pl.ds(lo, slab)])

    return mask_kernel, m_pad


def _select_body(x_ref, m_ref, v_ref, o_ref):
    o_ref[...] = jnp.where(m_ref[...] != 0.0, v_ref[0, 0], x_ref[...])


def _select_tc(x, mask2d, value_f32, blk_rows: int):
    num_rows, d = x.shape
    grid = (pl.cdiv(num_rows, blk_rows),)
    return pl.pallas_call(
        _select_body,
        grid=grid,
        in_specs=[
            pl.BlockSpec((blk_rows, d), lambda i: (i, 0)),
            pl.BlockSpec((blk_rows, 1), lambda i: (i, 0)),
            pl.BlockSpec(memory_space=pltpu.SMEM),
        ],
        out_specs=pl.BlockSpec((blk_rows, d), lambda i: (i, 0)),
        out_shape=jax.ShapeDtypeStruct((num_rows, d), x.dtype),
    )(x, mask2d, value_f32)


def kernel(x, dim, index, value):
    num_rows, _ = x.shape
    num_idx = index.shape[0]
    idx32 = index.astype(jnp.int32)

    mask_fn, m_pad = _build_mask_sc(num_rows, num_idx)
    mask = mask_fn(idx32)
    mask2d = mask.reshape(m_pad, 1)

    value_f32 = jnp.full((1, 1), value, dtype=jnp.float32)
    return _select_tc(x, mask2d, value_f32, blk_rows=20000)


# speedup vs baseline: 1.0132x; 1.0132x over previous
import functools

import jax
import jax.numpy as jnp
from jax import lax
from jax.experimental import pallas as pl
from jax.experimental.pallas import tpu as pltpu
from jax.experimental.pallas import tpu_sc as plsc

_NC = 2
_NS = 16
_NW = _NC * _NS
_L = 16


def _build_mask_sc(num_rows: int, num_idx: int):
    slab = ((num_rows + _NW - 1) // _NW + _L - 1) // _L * _L
    m_pad = slab * _NW
    mesh = plsc.VectorSubcoreMesh(core_axis_name="c", subcore_axis_name="s")

    @functools.partial(
        pl.kernel,
        out_type=jax.ShapeDtypeStruct((m_pad,), jnp.float32),
        mesh=mesh,
        scratch_types=[
            pltpu.VMEM((num_idx,), jnp.int32),
            pltpu.VMEM((slab,), jnp.float32),
        ],
        compiler_params=pltpu.CompilerParams(needs_layout_passes=False),
    )
    def mask_kernel(idx_hbm, mask_hbm, idx_v, slab_v):
        wid = lax.axis_index("s") * _NC + lax.axis_index("c")
        lo = wid * slab
        pltpu.sync_copy(idx_hbm, idx_v)

        zeros = jnp.zeros((_L,), jnp.float32)

        def zero_body(i, carry):
            slab_v[pl.ds(i * _L, _L)] = zeros
            return carry

        lax.fori_loop(0, slab // _L, zero_body, 0)

        ones = jnp.ones((_L,), jnp.float32)

        def scan_body(i, carry):
            v = idx_v[pl.ds(i * _L, _L)]
            m = (v >= lo) & (v < lo + slab)
            rel = jnp.where(m, v - lo, 0)
            plsc.store_scatter(slab_v, (rel,), ones, mask=m)
            return carry

        lax.fori_loop(0, num_idx // _L, scan_body, 0)

        pltpu.sync_copy(slab_v, mask_hbm.at[pl.ds(lo, slab)])

    return mask_kernel, m_pad


def _select_body(x_ref, m_ref, v_ref, o_ref):
    o_ref[...] = jnp.where(m_ref[...] != 0.0, v_ref[0, 0], x_ref[...])


def _select_tc(x, mask2d, value_f32, blk_rows: int):
    num_rows, d = x.shape
    grid = (pl.cdiv(num_rows, blk_rows),)
    return pl.pallas_call(
        _select_body,
        grid=grid,
        in_specs=[
            pl.BlockSpec((blk_rows, d), lambda i: (i, 0)),
            pl.BlockSpec((blk_rows, 1), lambda i: (i, 0)),
            pl.BlockSpec(memory_space=pltpu.SMEM),
        ],
        out_specs=pl.BlockSpec((blk_rows, d), lambda i: (i, 0)),
        out_shape=jax.ShapeDtypeStruct((num_rows, d), x.dtype),
    )(x, mask2d, value_f32)


def kernel(x, dim, index, value):
    num_rows, _ = x.shape
    num_idx = index.shape[0]
    idx32 = index.astype(jnp.int32)

    mask_fn, m_pad = _build_mask_sc(num_rows, num_idx)
    mask = mask_fn(idx32)
    mask2d = mask.reshape(m_pad, 1)

    value_f32 = jnp.full((1, 1), value, dtype=jnp.float32)
    return _select_tc(x, mask2d, value_f32, blk_rows=12504)


# speedup vs baseline: 1.1084x; 1.0940x over previous
import functools

import jax
import jax.numpy as jnp
from jax import lax
from jax.experimental import pallas as pl
from jax.experimental.pallas import tpu as pltpu
from jax.experimental.pallas import tpu_sc as plsc

_NC = 2
_NS = 16
_NW = _NC * _NS
_L = 16


def _index_fill_sc(m_rows: int, d: int, b_idx: int):
    ch = 128
    cs = 16
    nfull = m_rows // ch
    kfull = nfull // _NW
    extra = nfull - kfull * _NW
    tail = m_rows - nfull * ch
    tail_owner = nfull % _NW
    nbuf = 4
    nvec = b_idx // _L
    lane_cap = nvec + cs
    scan_per_k = -(-nvec // kfull)
    mesh = plsc.VectorSubcoreMesh(core_axis_name="c", subcore_axis_name="s")

    @functools.partial(
        pl.kernel,
        out_type=jax.ShapeDtypeStruct((m_rows, d), jnp.float32),
        mesh=mesh,
        scratch_types=(
            [pltpu.VMEM((ch, d), jnp.float32) for _ in range(nbuf)]
            + [
                pltpu.VMEM((b_idx,), jnp.int32),
                pltpu.VMEM((_L * lane_cap,), jnp.int32),
                pltpu.VMEM((cs, d), jnp.float32),
            ]
            + [pltpu.SemaphoreType.DMA for _ in range(2 * nbuf + 1)]
        ),
        compiler_params=pltpu.CompilerParams(needs_layout_passes=False),
    )
    def fill_kernel(x_hbm, idx_hbm, val_hbm, out_hbm, *rest):
        bufs = rest[:nbuf]
        idx_v, list_v, val_v = rest[nbuf:nbuf + 3]
        sems_in = rest[nbuf + 3:nbuf + 3 + nbuf]
        sems_out = rest[nbuf + 3 + nbuf:nbuf + 3 + 2 * nbuf]
        scat_sem = rest[-1]
        wid = lax.axis_index("s") * _NC + lax.axis_index("c")

        pltpu.sync_copy(idx_hbm, idx_v)
        pltpu.sync_copy(val_hbm, val_v)

        lane_base = lax.iota(jnp.int32, _L) * lane_cap

        def row0(k):
            return (wid + k * _NW) * ch

        def start_in(k):
            b = k % nbuf
            c = pltpu.make_async_copy(
                x_hbm.at[pl.ds(row0(k), ch)], bufs[b], sems_in[b])
            c.start()
            return c

        def start_out(k):
            b = k % nbuf
            c = pltpu.make_async_copy(
                bufs[b], out_hbm.at[pl.ds(row0(k), ch)], sems_out[b])
            c.start()
            return c

        def scan_body(i, carry):
            cnt_l, lastv = carry
            v = idx_v[pl.ds(i * _L, _L)]
            g = lax.shift_right_logical(v, 7)
            m = (g & (_NW - 1)) == wid
            plsc.store_scatter(list_v, (lane_base + cnt_l,), v, mask=m)
            return cnt_l + m.astype(jnp.int32), jnp.where(m, v, lastv)

        carry = (jnp.zeros((_L,), jnp.int32), jnp.full((_L,), -1, jnp.int32))

        for k in range(kfull):
            pltpu.sync_copy(x_hbm.at[pl.ds(row0(k), ch)], bufs[k % nbuf])
            pltpu.sync_copy(bufs[k % nbuf], out_hbm.at[pl.ds(row0(k), ch)])
            lo = min(k * scan_per_k, nvec)
            hi = min((k + 1) * scan_per_k, nvec)
            if lo < hi:
                carry = lax.fori_loop(lo, hi, scan_body, carry)

        if extra > 0:
            @pl.when(wid < extra)
            def _extra_chunk():
                r = (kfull * _NW + wid) * ch
                pltpu.sync_copy(x_hbm.at[pl.ds(r, ch)], bufs[0])
                pltpu.sync_copy(bufs[0], out_hbm.at[pl.ds(r, ch)])

        if tail > 0:
            @pl.when(wid == tail_owner)
            def _tail_chunk():
                r = nfull * ch
                pltpu.sync_copy(x_hbm.at[pl.ds(r, tail)],
                                bufs[1].at[pl.ds(0, tail)])
                pltpu.sync_copy(bufs[1].at[pl.ds(0, tail)],
                                out_hbm.at[pl.ds(r, tail)])

        cnt_l, lastv = carry

        lane_iota = lax.iota(jnp.int32, _L)
        for lane in range(_L):
            cnt = cnt_l[lane]
            pad = jnp.broadcast_to(lastv[lane], (_L,))
            plsc.store_scatter(
                list_v, (lane * lane_cap + cnt + lane_iota,), pad)
            n_grp = lax.shift_right_logical(cnt + (cs - 1), 4)

            def fire(q, c, _lane=lane):
                pltpu.sync_copy(
                    val_v,
                    out_hbm.at[list_v.at[pl.ds(_lane * lane_cap + q * cs, cs)]])
                return c

            lax.fori_loop(0, n_grp, fire, 0)

    return fill_kernel


def kernel(x, dim, index, value):
    m_rows, d = x.shape
    b_idx = index.shape[0]
    idx32 = index.astype(jnp.int32)
    val_arr = jnp.full((16, d), value, dtype=jnp.float32)
    fn = _index_fill_sc(m_rows, d, b_idx)
    return fn(x, idx32, val_arr)


# speedup vs baseline: 1.2358x; 1.1150x over previous
import functools

import jax
import jax.numpy as jnp
from jax import lax
from jax.experimental import pallas as pl
from jax.experimental.pallas import tpu as pltpu
from jax.experimental.pallas import tpu_sc as plsc

_NC = 2
_NS = 16
_NW = _NC * _NS
_L = 16


def _index_fill_sc(m_rows: int, d: int, b_idx: int):
    ch = 128
    cs = 16
    nfull = m_rows // ch
    kfull = nfull // _NW
    extra = nfull - kfull * _NW
    tail = m_rows - nfull * ch
    tail_owner = nfull % _NW
    nbuf = 4
    nvec = b_idx // _L
    lane_cap = nvec + cs
    scan_per_k = -(-nvec // kfull)
    chw = ch * d
    mesh = plsc.VectorSubcoreMesh(core_axis_name="c", subcore_axis_name="s")

    @functools.partial(
        pl.kernel,
        out_type=jax.ShapeDtypeStruct((m_rows, d), jnp.float32),
        mesh=mesh,
        scratch_types=(
            [pltpu.VMEM((ch, d), jnp.float32) for _ in range(nbuf)]
            + [
                pltpu.VMEM((b_idx,), jnp.int32),
                pltpu.VMEM((_L * lane_cap,), jnp.int32),
                pltpu.VMEM((cs, d), jnp.float32),
            ]
            + [pltpu.SemaphoreType.DMA for _ in range(2 * nbuf + 1)]
        ),
        compiler_params=pltpu.CompilerParams(needs_layout_passes=False),
    )
    def fill_kernel(x_hbm, idx_hbm, val_hbm, out_hbm, *rest):
        bufs = rest[:nbuf]
        idx_v, list_v, val_v = rest[nbuf:nbuf + 3]
        sems_in = rest[nbuf + 3:nbuf + 3 + nbuf]
        sems_out = rest[nbuf + 3 + nbuf:nbuf + 3 + 2 * nbuf]
        scat_sem = rest[-1]
        wid = lax.axis_index("s") * _NC + lax.axis_index("c")

        pltpu.sync_copy(idx_hbm, idx_v)
        pltpu.sync_copy(val_hbm, val_v)

        lane_base = lax.iota(jnp.int32, _L) * lane_cap

        def row0(k):
            return (wid + k * _NW) * ch

        def start_in(k):
            b = k % nbuf
            pltpu.make_async_copy(
                x_hbm.at[pl.ds(row0(k), ch)], bufs[b], sems_in[b]).start()

        def start_out(k):
            b = k % nbuf
            pltpu.make_async_copy(
                bufs[b], out_hbm.at[pl.ds(row0(k), ch)], sems_out[b]).start()

        def scan_body(i, carry):
            cnt_l, lastv = carry
            v = idx_v[pl.ds(i * _L, _L)]
            g = lax.shift_right_logical(v, 7)
            m = (g & (_NW - 1)) == wid
            plsc.store_scatter(list_v, (lane_base + cnt_l,), v, mask=m)
            return cnt_l + m.astype(jnp.int32), jnp.where(m, v, lastv)

        carry = (jnp.zeros((_L,), jnp.int32), jnp.full((_L,), -1, jnp.int32))

        out_d = [None] * kfull
        for k in range(kfull):
            b = k % nbuf
            if k >= nbuf:
                out_d[k - nbuf].wait()
            pltpu.sync_copy(x_hbm.at[pl.ds(row0(k), ch)], bufs[b])
            c = pltpu.make_async_copy(
                bufs[b], out_hbm.at[pl.ds(row0(k), ch)], sems_out[b])
            c.start()
            out_d[k] = c
            lo = min(k * scan_per_k, nvec)
            hi = min((k + 1) * scan_per_k, nvec)
            if lo < hi:
                carry = lax.fori_loop(lo, hi, scan_body, carry)
        for k in range(max(0, kfull - nbuf), kfull):
            out_d[k].wait()

        if extra > 0:
            @pl.when(wid < extra)
            def _extra_chunk():
                r = (kfull * _NW + wid) * ch
                pltpu.sync_copy(x_hbm.at[pl.ds(r, ch)], bufs[0])
                pltpu.sync_copy(bufs[0], out_hbm.at[pl.ds(r, ch)])

        if tail > 0:
            @pl.when(wid == tail_owner)
            def _tail_chunk():
                r = nfull * ch
                pltpu.sync_copy(x_hbm.at[pl.ds(r, tail)],
                                bufs[1].at[pl.ds(0, tail)])
                pltpu.sync_copy(bufs[1].at[pl.ds(0, tail)],
                                out_hbm.at[pl.ds(r, tail)])

        cnt_l, lastv = carry
        lane_iota = lax.iota(jnp.int32, _L)
        n_grp_l = []
        for lane in range(_L):
            cnt = cnt_l[lane]
            pad = jnp.broadcast_to(lastv[lane], (_L,))
            plsc.store_scatter(
                list_v, (lane * lane_cap + cnt + lane_iota,), pad)
            n_grp = lax.shift_right_logical(cnt + (cs - 1), 4)
            n_grp_l.append(n_grp)

            def fire(q, c, _lane=lane):
                pltpu.sync_copy(
                    val_v,
                    out_hbm.at[list_v.at[pl.ds(_lane * lane_cap + q * cs, cs)]])
                return c

            lax.fori_loop(0, n_grp, fire, 0)
        del n_grp_l

    return fill_kernel


def kernel(x, dim, index, value):
    m_rows, d = x.shape
    b_idx = index.shape[0]
    idx32 = index.astype(jnp.int32)
    val_arr = jnp.full((16, d), value, dtype=jnp.float32)
    fn = _index_fill_sc(m_rows, d, b_idx)
    return fn(x, idx32, val_arr)


# speedup vs baseline: 1.2391x; 1.0026x over previous
import functools

import jax
import jax.numpy as jnp
from jax import lax
from jax.experimental import pallas as pl
from jax.experimental.pallas import tpu as pltpu
from jax.experimental.pallas import tpu_sc as plsc

_NC = 2
_NS = 16
_NW = _NC * _NS
_L = 16


def _index_fill_sc(m_rows: int, d: int, b_idx: int):
    ch = 128
    cs = 16
    nfull = m_rows // ch
    kfull = nfull // _NW
    extra = nfull - kfull * _NW
    tail = m_rows - nfull * ch
    tail_owner = nfull % _NW
    nbuf = 4
    nvec = b_idx // _L
    lane_cap = nvec + cs
    scan_per_k = -(-nvec // kfull)
    chw = ch * d
    mesh = plsc.VectorSubcoreMesh(core_axis_name="c", subcore_axis_name="s")

    @functools.partial(
        pl.kernel,
        out_type=jax.ShapeDtypeStruct((m_rows, d), jnp.float32),
        mesh=mesh,
        scratch_types=(
            [pltpu.VMEM((ch, d), jnp.float32) for _ in range(nbuf)]
            + [
                pltpu.VMEM((b_idx,), jnp.int32),
                pltpu.VMEM((_L * lane_cap,), jnp.int32),
                pltpu.VMEM((cs, d), jnp.float32),
            ]
            + [pltpu.SemaphoreType.DMA for _ in range(2 * nbuf + 1)]
        ),
        compiler_params=pltpu.CompilerParams(needs_layout_passes=False),
    )
    def fill_kernel(x_hbm, idx_hbm, val_hbm, out_hbm, *rest):
        bufs = rest[:nbuf]
        idx_v, list_v, val_v = rest[nbuf:nbuf + 3]
        sems_in = rest[nbuf + 3:nbuf + 3 + nbuf]
        sems_out = rest[nbuf + 3 + nbuf:nbuf + 3 + 2 * nbuf]
        scat_sem = rest[-1]
        wid = lax.axis_index("s") * _NC + lax.axis_index("c")

        pltpu.sync_copy(idx_hbm, idx_v)
        pltpu.sync_copy(val_hbm, val_v)

        lane_base = lax.iota(jnp.int32, _L) * lane_cap

        def row0(k):
            return (wid + k * _NW) * ch

        def scan_body(i, carry):
            cnt_l, lastv = carry
            v = idx_v[pl.ds(i * _L, _L)]
            g = lax.shift_right_logical(v, 7)
            m = (g & (_NW - 1)) == wid
            plsc.store_scatter(list_v, (lane_base + cnt_l,), v, mask=m)
            return cnt_l + m.astype(jnp.int32), jnp.where(m, v, lastv)

        carry = (jnp.zeros((_L,), jnp.int32), jnp.full((_L,), -1, jnp.int32))

        out_d = [None] * kfull
        for k in range(kfull):
            b = k % nbuf
            if k >= nbuf:
                out_d[k - nbuf].wait()
            pltpu.sync_copy(x_hbm.at[pl.ds(row0(k), ch)], bufs[b])
            c = pltpu.make_async_copy(
                bufs[b], out_hbm.at[pl.ds(row0(k), ch)], sems_out[b])
            c.start()
            out_d[k] = c
            lo = min(k * scan_per_k, nvec)
            hi = min((k + 1) * scan_per_k, nvec)
            if lo < hi:
                carry = lax.fori_loop(lo, hi, scan_body, carry)
        for k in range(max(0, kfull - nbuf), kfull):
            out_d[k].wait()

        if extra > 0:
            @pl.when(wid < extra)
            def _extra_chunk():
                r = (kfull * _NW + wid) * ch
                pltpu.sync_copy(x_hbm.at[pl.ds(r, ch)], bufs[0])
                pltpu.sync_copy(bufs[0], out_hbm.at[pl.ds(r, ch)])

        if tail > 0:
            @pl.when(wid == tail_owner)
            def _tail_chunk():
                r = nfull * ch
                pltpu.sync_copy(x_hbm.at[pl.ds(r, tail)],
                                bufs[1].at[pl.ds(0, tail)])
                pltpu.sync_copy(bufs[1].at[pl.ds(0, tail)],
                                out_hbm.at[pl.ds(r, tail)])

        cnt_l, lastv = carry
        lane_iota = lax.iota(jnp.int32, _L)
        n_grp_l = []
        for lane in range(_L):
            cnt = cnt_l[lane]
            pad = jnp.broadcast_to(lastv[lane], (_L,))
            plsc.store_scatter(
                list_v, (lane * lane_cap + cnt + lane_iota,), pad)
            n_grp = lax.shift_right_logical(cnt + (cs - 1), 4)
            n_grp_l.append(n_grp)

            def fire(q, c, _lane=lane):
                pltpu.sync_copy(
                    val_v,
                    out_hbm.at[list_v.at[pl.ds(_lane * lane_cap + q * cs, cs)]])
                return c

            lax.fori_loop(0, n_grp, fire, 0)
        del n_grp_l

    return fill_kernel


def kernel(x, dim, index, value):
    m_rows, d = x.shape
    b_idx = index.shape[0]
    idx32 = index.astype(jnp.int32)
    val_arr = jnp.full((16, d), value, dtype=jnp.float32)
    fn = _index_fill_sc(m_rows, d, b_idx)
    return fn(x, idx32, val_arr)
